# baseline (speedup 1.0000x reference)
import functools

import jax
import jax.numpy as jnp
from jax import lax
from jax.experimental import pallas as pl
from jax.experimental.pallas import tpu as pltpu

_H_RATIO = 0.33


def _tree_sum(parts):
    while len(parts) > 1:
        nxt = [a + b for a, b in zip(parts[::2], parts[1::2])]
        if len(parts) % 2:
            nxt.append(parts[-1])
        parts = nxt
    return parts[0]


def _bdt_block(x_ref, o_ref, *, h, w, rh):
    xb = x_ref[...]
    s_blk, c, hw = xb.shape
    pad = 256
    ngrp = pad // w

    nchunk = 8
    step = c // nchunk
    parts = [
        jnp.sum(xb[:, i * step:(i + 1) * step, :] ** 2, axis=1)
        for i in range(nchunk)
    ]
    e = _tree_sum(parts)

    e = jnp.concatenate(
        [e, jnp.full((s_blk, pad - hw), -1.0, e.dtype)], axis=1)

    lane = lax.broadcasted_iota(jnp.int32, (s_blk, pad), 1)

    m = e
    s = 1
    while s < w:
        up = pltpu.roll(m, pad - s, axis=1)
        dn = pltpu.roll(m, s, axis=1)
        m = jnp.maximum(m, jnp.where((lane % (2 * s)) < s, up, dn))
        s *= 2

    g = lane // w
    beats = []
    for d in range(1, ngrp):
        md = pltpu.roll(m, pad - w * d, axis=1)
        gd = g + d
        gd = jnp.where(gd >= ngrp, gd - ngrp, gd)
        beat = (md > m) | ((md == m) & (gd > g))
        beats.append(beat.astype(jnp.int32))
    rank = _tree_sum(beats)

    keep = (rank >= rh).astype(xb.dtype)[:, :hw]
    o_ref[:, :, 0:hw] = xb * keep[:, None, :]


def kernel(x):
    b, c, h, w = x.shape
    rh = int(round(_H_RATIO * h))
    hw = h * w
    s_blk = 4
    x3 = x.reshape(b, c, hw)

    body = functools.partial(_bdt_block, h=h, w=w, rh=rh)
    out = pl.pallas_call(
        body,
        grid=(b // s_blk,),
        in_specs=[pl.BlockSpec((s_blk, c, hw), lambda i: (i, 0, 0))],
        out_specs=pl.BlockSpec((s_blk, c, 256), lambda i: (i, 0, 0)),
        out_shape=jax.ShapeDtypeStruct((b, c, 256), x.dtype),
    )(x3)
    return out[:, :, :hw].reshape(b, c, h, w)
